# baseline (speedup 1.0000x reference)
import jax
import jax.numpy as jnp
from jax.experimental import pallas as pl
from jax.experimental.pallas import tpu as pltpu

N = 8192
BM = 512
BK = 2048
NK = N // BK


def _sage_kernel(adj_ref, x_ref, w_ref, b_ref, o_ref, acc_ref, deg_ref):
    k = pl.program_id(1)
    a = adj_ref[...]

    @pl.when(k == 0)
    def _():
        acc_ref[...] = jnp.zeros_like(acc_ref)
        deg_ref[...] = jnp.zeros_like(deg_ref)

    acc_ref[...] += jnp.dot(a, x_ref[...], preferred_element_type=jnp.float32)
    deg_ref[...] += jnp.sum(a, axis=1, keepdims=True)

    @pl.when(k == NK - 1)
    def _():
        h = jnp.dot(acc_ref[...], w_ref[...].T,
                    preferred_element_type=jnp.float32)
        deg = deg_ref[...]
        out = (h + deg * b_ref[...]) / (deg + 1e-6)
        o_ref[...] = jnp.maximum(out, 0.0)


def kernel(x, adj, W, b):
    n, d_in = x.shape
    d_out = W.shape[0]
    b2 = b.reshape(1, d_out)
    return pl.pallas_call(
        _sage_kernel,
        grid=(n // BM, NK),
        in_specs=[
            pl.BlockSpec((BM, BK), lambda i, k: (i, k)),
            pl.BlockSpec((BK, d_in), lambda i, k: (k, 0)),
            pl.BlockSpec((d_out, d_in), lambda i, k: (0, 0)),
            pl.BlockSpec((1, d_out), lambda i, k: (0, 0)),
        ],
        out_specs=pl.BlockSpec((BM, d_out), lambda i, k: (i, 0)),
        out_shape=jax.ShapeDtypeStruct((n, d_out), jnp.float32),
        scratch_shapes=[
            pltpu.VMEM((BM, d_in), jnp.float32),
            pltpu.VMEM((BM, 1), jnp.float32),
        ],
        compiler_params=pltpu.CompilerParams(
            dimension_semantics=("parallel", "arbitrary"),
        ),
    )(adj, x, W, b2)


# speedup vs baseline: 1.4706x; 1.4706x over previous
import jax
import jax.numpy as jnp
from jax.experimental import pallas as pl
from jax.experimental.pallas import tpu as pltpu

N = 8192
BM = 512


def _sage_kernel(adj_ref, x_ref, w_ref, b_ref, o_ref):
    a = adj_ref[...]
    ax = jnp.dot(a, x_ref[...], preferred_element_type=jnp.float32)
    h = jnp.dot(ax, w_ref[...].T, preferred_element_type=jnp.float32)
    deg = jnp.sum(a, axis=1, keepdims=True)
    out = (h + deg * b_ref[...]) / (deg + 1e-6)
    o_ref[...] = jnp.maximum(out, 0.0)


def kernel(x, adj, W, b):
    n, d_in = x.shape
    d_out = W.shape[0]
    b2 = b.reshape(1, d_out)
    return pl.pallas_call(
        _sage_kernel,
        grid=(n // BM,),
        in_specs=[
            pl.BlockSpec((BM, n), lambda i: (i, 0)),
            pl.BlockSpec((n, d_in), lambda i: (0, 0)),
            pl.BlockSpec((d_out, d_in), lambda i: (0, 0)),
            pl.BlockSpec((1, d_out), lambda i: (0, 0)),
        ],
        out_specs=pl.BlockSpec((BM, d_out), lambda i: (i, 0)),
        out_shape=jax.ShapeDtypeStruct((n, d_out), jnp.float32),
        compiler_params=pltpu.CompilerParams(
            dimension_semantics=("parallel",),
        ),
    )(adj, x, W, b2)
